# baseline (speedup 1.0000x reference)
import functools

import jax
import jax.numpy as jnp
from jax import lax
from jax.experimental import pallas as pl
from jax.experimental.pallas import tpu as pltpu
from jax.experimental.pallas import tpu_sc as plsc

_EPS = 1e-12
_H = 128
_G = _H // 16
_C = 128
_NB = 5


def _rsqrt(a):
    i = lax.bitcast_convert_type(a, jnp.int32)
    i = jnp.int32(0x5F3759DF) - (i >> 1)
    y = lax.bitcast_convert_type(i, jnp.float32)
    for _ in range(3):
        y = y * (1.5 - 0.5 * a * y * y)
    return y


def _make_sc_kernel(N, S, n_workers):
    toks_per_w = N // n_workers
    n_chunks = toks_per_w // _C
    assert n_chunks % _NB == 0
    mesh = plsc.VectorSubcoreMesh(core_axis_name="c", subcore_axis_name="s")

    @functools.partial(
        pl.kernel,
        out_type=jax.ShapeDtypeStruct((N, _H), jnp.float32),
        mesh=mesh,
        compiler_params=pltpu.CompilerParams(needs_layout_passes=False),
        scratch_types=[
            [pltpu.VMEM((_C,), jnp.int32) for _ in range(_NB)],
            [pltpu.VMEM((_C, _H), jnp.float32) for _ in range(_NB)],
            [pltpu.VMEM((_C + 16,), jnp.float32) for _ in range(_NB)],
            pltpu.VMEM((S, _H), jnp.float32),
            pltpu.VMEM((_H,), jnp.float32),
            pltpu.VMEM((_H,), jnp.float32),
            pltpu.VMEM((_H,), jnp.float32),
            pltpu.VMEM((_H,), jnp.float32),
            [pltpu.SemaphoreType.DMA for _ in range(_NB)],
            [pltpu.SemaphoreType.DMA for _ in range(_NB)],
            [pltpu.SemaphoreType.DMA for _ in range(_NB)],
            [pltpu.SemaphoreType.DMA for _ in range(_NB)],
        ],
    )
    def k(ids_hbm, vals_hbm, gene_hbm, pos_hbm, w_hbm, b_hbm, g_hbm,
          beta_hbm, out_hbm, ids_b, rows_b, vals_b, pos_v,
          w_v, b_v, g_v, beta_v, gsem_b, ssem_b, isem_b, vsem_b):
        nc = plsc.get_sparse_core_info().num_cores
        wid = lax.axis_index("s") * nc + lax.axis_index("c")
        w_base = wid * toks_per_w

        pltpu.sync_copy(pos_hbm.at[pl.ds(0, S)], pos_v)
        pltpu.sync_copy(w_hbm, w_v)
        pltpu.sync_copy(b_hbm, b_v)
        pltpu.sync_copy(g_hbm, g_v)
        pltpu.sync_copy(beta_hbm, beta_v)

        w_r = [w_v[pl.ds(16 * g, 16)] for g in range(_G)]
        b_r = [b_v[pl.ds(16 * g, 16)] for g in range(_G)]
        gm_r = [g_v[pl.ds(16 * g, 16)] for g in range(_G)]
        bt_r = [beta_v[pl.ds(16 * g, 16)] for g in range(_G)]

        @plsc.parallel_loop(0, S, step=1, unroll=4)
        def _fold_b(s):
            for g in range(_G):
                pos_v[s, pl.ds(16 * g, 16)] = (
                    pos_v[s, pl.ds(16 * g, 16)] + b_r[g])

        def stage_io(chunk, bi):
            base = w_base + chunk * _C
            pltpu.async_copy(ids_hbm.at[pl.ds(base, _C)], ids_b[bi],
                             isem_b[bi])
            pltpu.async_copy(vals_hbm.at[pl.ds(base, _C)],
                             vals_b[bi].at[pl.ds(0, _C)], vsem_b[bi])

        def fire_gather(chunk, bi):
            base = w_base + chunk * _C
            pltpu.make_async_copy(ids_hbm.at[pl.ds(base, _C)], ids_b[bi],
                                  isem_b[bi]).wait()

            @pl.when(chunk >= _NB)
            def _():
                prev = w_base + (chunk - _NB) * _C
                pltpu.make_async_copy(
                    rows_b[bi], out_hbm.at[pl.ds(prev, _C)],
                    ssem_b[bi]).wait()

            pltpu.async_copy(gene_hbm.at[ids_b[bi]], rows_b[bi],
                             gsem_b[bi])

        def compute(chunk, bi):
            rows_v = rows_b[bi]
            vals_v = vals_b[bi]
            base = w_base + chunk * _C
            pltpu.make_async_copy(
                vals_hbm.at[pl.ds(base, _C)],
                vals_v.at[pl.ds(0, _C)], vsem_b[bi]).wait()
            pltpu.make_async_copy(
                gene_hbm.at[ids_b[bi]], rows_v, gsem_b[bi]).wait()
            @plsc.parallel_loop(0, _C, step=1, unroll=10)
            def tok_body(t):
                s = (base + t) % S
                v_t = jnp.full((16,), vals_v[pl.ds(t, 16)][0], jnp.float32)
                xs = []
                acc = None
                acc2 = None
                for g in range(_G):
                    x = (rows_v[t, pl.ds(16 * g, 16)]
                         + pos_v[s, pl.ds(16 * g, 16)]
                         + v_t * w_r[g])
                    xs.append(x)
                    acc = x if acc is None else acc + x
                    acc2 = x * x if acc2 is None else acc2 + x * x
                mean = jnp.sum(acc, axis=0) * (1.0 / _H)
                var = jnp.sum(acc2, axis=0) * (1.0 / _H) - mean * mean
                rstd = _rsqrt(var + _EPS)
                mean_v = jnp.full((16,), mean, jnp.float32)
                rstd_v = jnp.full((16,), rstd, jnp.float32)
                for g in range(_G):
                    y = (xs[g] - mean_v) * rstd_v * gm_r[g] + bt_r[g]
                    rows_v[t, pl.ds(16 * g, 16)] = y

            pltpu.async_copy(rows_v, out_hbm.at[pl.ds(base, _C)],
                             ssem_b[bi])

        stage_io(0, 0)
        fire_gather(0, 0)
        stage_io(1, 1)

        def outer(cg, _):
            for j in range(_NB):
                chunk = cg * _NB + j

                @pl.when(chunk + 2 < n_chunks)
                def _():
                    stage_io(chunk + 2, (j + 2) % _NB)

                @pl.when(chunk + 1 < n_chunks)
                def _():
                    fire_gather(chunk + 1, (j + 1) % _NB)

                compute(chunk, j)
            return ()

        lax.fori_loop(0, n_chunks // _NB, outer, (), unroll=False)

        for last in range(n_chunks - _NB, n_chunks):
            basel = w_base + last * _C
            pltpu.make_async_copy(
                rows_b[last % _NB], out_hbm.at[pl.ds(basel, _C)],
                ssem_b[last % _NB]).wait()

    return k


def kernel(input_ids, values, gene_table, pos_table, expr_w, expr_b,
           ln_gamma, ln_beta):
    B, S = input_ids.shape
    N = B * S
    ids = input_ids.reshape(N).astype(jnp.int32)
    vals = values.reshape(N)
    k = _make_sc_kernel(N, S, 32)
    out = k(ids, vals, gene_table, pos_table, expr_w, expr_b, ln_gamma,
            ln_beta)
    return out.reshape(B, S, _H)


# speedup vs baseline: 1.0467x; 1.0467x over previous
import functools

import jax
import jax.numpy as jnp
from jax import lax
from jax.experimental import pallas as pl
from jax.experimental.pallas import tpu as pltpu
from jax.experimental.pallas import tpu_sc as plsc

_EPS = 1e-12
_H = 128
_G = _H // 16
_C = 128
_NB = 5


def _rsqrt(a):
    i = lax.bitcast_convert_type(a, jnp.int32)
    i = jnp.int32(0x5F3759DF) - (i >> 1)
    y = lax.bitcast_convert_type(i, jnp.float32)
    for _ in range(3):
        y = y * (1.5 - 0.5 * a * y * y)
    return y


def _make_sc_kernel(N, S, n_workers):
    toks_per_w = N // n_workers
    n_chunks = toks_per_w // _C
    assert n_chunks % _NB == 0
    mesh = plsc.VectorSubcoreMesh(core_axis_name="c", subcore_axis_name="s")

    @functools.partial(
        pl.kernel,
        out_type=jax.ShapeDtypeStruct((N, _H), jnp.float32),
        mesh=mesh,
        compiler_params=pltpu.CompilerParams(needs_layout_passes=False),
        scratch_types=[
            [pltpu.VMEM((_C,), jnp.int32) for _ in range(_NB)],
            [pltpu.VMEM((_C, _H), jnp.float32) for _ in range(_NB)],
            [pltpu.VMEM((_C + 16,), jnp.float32) for _ in range(_NB)],
            pltpu.VMEM((S, _H), jnp.float32),
            pltpu.VMEM((_H,), jnp.float32),
            pltpu.VMEM((_H,), jnp.float32),
            pltpu.VMEM((_H,), jnp.float32),
            pltpu.VMEM((_H,), jnp.float32),
            [pltpu.SemaphoreType.DMA for _ in range(_NB)],
            [pltpu.SemaphoreType.DMA for _ in range(_NB)],
            [pltpu.SemaphoreType.DMA for _ in range(_NB)],
            [pltpu.SemaphoreType.DMA for _ in range(_NB)],
        ],
    )
    def k(ids_hbm, vals_hbm, gene_hbm, pos_hbm, w_hbm, b_hbm, g_hbm,
          beta_hbm, out_hbm, ids_b, rows_b, vals_b, pos_v,
          w_v, b_v, g_v, beta_v, gsem_b, ssem_b, isem_b, vsem_b):
        nc = plsc.get_sparse_core_info().num_cores
        wid = lax.axis_index("s") * nc + lax.axis_index("c")
        w_base = wid * toks_per_w

        pltpu.sync_copy(pos_hbm.at[pl.ds(0, S)], pos_v)
        pltpu.sync_copy(w_hbm, w_v)
        pltpu.sync_copy(b_hbm, b_v)
        pltpu.sync_copy(g_hbm, g_v)
        pltpu.sync_copy(beta_hbm, beta_v)

        w_r = [w_v[pl.ds(16 * g, 16)] for g in range(_G)]
        b_r = [b_v[pl.ds(16 * g, 16)] for g in range(_G)]
        gm_r = [g_v[pl.ds(16 * g, 16)] for g in range(_G)]
        bt_r = [beta_v[pl.ds(16 * g, 16)] for g in range(_G)]

        @plsc.parallel_loop(0, S, step=1, unroll=4)
        def _fold_b(s):
            for g in range(_G):
                pos_v[s, pl.ds(16 * g, 16)] = (
                    pos_v[s, pl.ds(16 * g, 16)] + b_r[g])

        def stage_io(chunk, bi):
            base = w_base + chunk * _C
            pltpu.async_copy(ids_hbm.at[pl.ds(base, _C)], ids_b[bi],
                             isem_b[bi])
            pltpu.async_copy(vals_hbm.at[pl.ds(base, _C)],
                             vals_b[bi].at[pl.ds(0, _C)], vsem_b[bi])

        def fire_gather(chunk, bi):
            base = w_base + chunk * _C
            pltpu.make_async_copy(ids_hbm.at[pl.ds(base, _C)], ids_b[bi],
                                  isem_b[bi]).wait()

            @pl.when(chunk >= _NB)
            def _():
                prev = w_base + (chunk - _NB) * _C
                pltpu.make_async_copy(
                    rows_b[bi], out_hbm.at[pl.ds(prev, _C)],
                    ssem_b[bi]).wait()

            pltpu.async_copy(gene_hbm.at[ids_b[bi]], rows_b[bi],
                             gsem_b[bi])

        def compute(chunk, bi):
            rows_v = rows_b[bi]
            vals_v = vals_b[bi]
            base = w_base + chunk * _C
            pltpu.make_async_copy(
                vals_hbm.at[pl.ds(base, _C)],
                vals_v.at[pl.ds(0, _C)], vsem_b[bi]).wait()
            pltpu.make_async_copy(
                gene_hbm.at[ids_b[bi]], rows_v, gsem_b[bi]).wait()
            @plsc.parallel_loop(0, _C, step=1, unroll=8)
            def tok_body(t):
                s = (base + t) % S
                v_t = jnp.full((16,), vals_v[pl.ds(t, 16)][0], jnp.float32)
                xs = []
                acc = None
                acc2 = None
                for g in range(_G):
                    x = (rows_v[t, pl.ds(16 * g, 16)]
                         + pos_v[s, pl.ds(16 * g, 16)]
                         + v_t * w_r[g])
                    xs.append(x)
                    acc = x if acc is None else acc + x
                    acc2 = x * x if acc2 is None else acc2 + x * x
                mean = jnp.sum(acc, axis=0) * (1.0 / _H)
                var = jnp.sum(acc2, axis=0) * (1.0 / _H) - mean * mean
                rstd = _rsqrt(var + _EPS)
                mean_v = jnp.full((16,), mean, jnp.float32)
                rstd_v = jnp.full((16,), rstd, jnp.float32)
                for g in range(_G):
                    y = (xs[g] - mean_v) * rstd_v * gm_r[g] + bt_r[g]
                    rows_v[t, pl.ds(16 * g, 16)] = y

            pltpu.async_copy(rows_v, out_hbm.at[pl.ds(base, _C)],
                             ssem_b[bi])

        stage_io(0, 0)
        fire_gather(0, 0)
        stage_io(1, 1)

        def outer(cg, _):
            for j in range(_NB):
                chunk = cg * _NB + j

                @pl.when(chunk + 2 < n_chunks)
                def _():
                    stage_io(chunk + 2, (j + 2) % _NB)

                @pl.when(chunk + 1 < n_chunks)
                def _():
                    fire_gather(chunk + 1, (j + 1) % _NB)

                compute(chunk, j)
            return ()

        lax.fori_loop(0, n_chunks // _NB, outer, (), unroll=False)

        for last in range(n_chunks - _NB, n_chunks):
            basel = w_base + last * _C
            pltpu.make_async_copy(
                rows_b[last % _NB], out_hbm.at[pl.ds(basel, _C)],
                ssem_b[last % _NB]).wait()

    return k


def kernel(input_ids, values, gene_table, pos_table, expr_w, expr_b,
           ln_gamma, ln_beta):
    B, S = input_ids.shape
    N = B * S
    ids = input_ids.reshape(N).astype(jnp.int32)
    vals = values.reshape(N)
    k = _make_sc_kernel(N, S, 32)
    out = k(ids, vals, gene_table, pos_table, expr_w, expr_b, ln_gamma,
            ln_beta)
    return out.reshape(B, S, _H)
